# baseline (speedup 1.0000x reference)
import jax
import jax.numpy as jnp
from jax.experimental import pallas as pl
from jax.experimental.pallas import tpu as pltpu

_BLK = 2048


def _body(x_ref, o_hbm, sem):
    i = pl.program_id(0)
    cp = pltpu.make_async_copy(x_ref, o_hbm.at[pl.ds(i * _BLK, _BLK)], sem)
    cp.start()
    cp.wait()


def kernel(x, question_mask, table_mask, column_mask):
    n, d = x.shape
    return pl.pallas_call(
        _body,
        grid=(n // _BLK,),
        in_specs=[pl.BlockSpec((_BLK, d), lambda i: (i, 0))],
        out_specs=pl.BlockSpec(memory_space=pl.ANY),
        out_shape=jax.ShapeDtypeStruct((n, d), x.dtype),
        scratch_shapes=[pltpu.SemaphoreType.DMA],
    )(x)


# speedup vs baseline: 1.1505x; 1.1505x over previous
import jax
import jax.numpy as jnp
from jax.experimental import pallas as pl


def _copy_body(x_ref, o_ref):
    o_ref[...] = x_ref[...]


def kernel(x, question_mask, table_mask, column_mask):
    n, d = x.shape
    bn, bd = 8192, 256
    return pl.pallas_call(
        _copy_body,
        grid=(n // bn, d // bd),
        in_specs=[pl.BlockSpec((bn, bd), lambda i, j: (i, j))],
        out_specs=pl.BlockSpec((bn, bd), lambda i, j: (i, j)),
        out_shape=jax.ShapeDtypeStruct((n, d), x.dtype),
    )(x)
